# baseline (speedup 1.0000x reference)
import jax
import jax.numpy as jnp
from jax import lax
from jax.experimental import pallas as pl
from jax.experimental.pallas import tpu as pltpu
from jax.experimental.pallas import tpu_sc as plsc

_D = 64
_W = 2 * _D
_B = 16384
_NC = 2
_NS = 16
_NW = _NC * _NS
_BPW = _B // _NW
_CHUNK = 128
_NCHUNK = _BPW // _CHUNK

_E = 100000
_EBLK = 128
_NFULL = _E // _EBLK
_EPAD = (_NFULL + 1) * _EBLK
_TPB = 25

_PARAMS = pltpu.CompilerParams(
    needs_layout_passes=False, use_tc_tiling_on_sc=True)
_MESH = dict(core_axis_name="c", subcore_axis_name="s")


def _fuse_kernel(re_t_h, im_t_h, tail_h, out_h, v3_a, v3_b, out_a, out_b,
                 sin_a, sin_b, sout_a, sout_b):
    wid = lax.axis_index("s") * _NC + lax.axis_index("c")
    v3 = (v3_a, v3_b)
    outv = (out_a, out_b)
    sin = (sin_a, sin_b)
    sout = (sout_a, sout_b)
    iota = lax.iota(jnp.int32, 16)
    avec = [(16 * k + iota) // 8 for k in range(8)]
    dvec = [(16 * k + iota) % 8 for k in range(8)]

    def blk(t):
        return wid + _NW * t

    def in_descs(t, p):
        b = blk(t)
        ds = []
        for a in range(8):
            src = re_t_h.at[pl.ds(8 * a, 8), pl.ds(_EBLK * b, _EBLK)]
            ds.append(pltpu.make_async_copy(src, v3[p].at[a], sin[p]))
        for a in range(8):
            src = im_t_h.at[pl.ds(8 * a, 8), pl.ds(_EBLK * b, _EBLK)]
            ds.append(pltpu.make_async_copy(src, v3[p].at[8 + a], sin[p]))
        return ds

    def out_desc(t, p):
        b = blk(t)
        return pltpu.make_async_copy(
            outv[p], out_h.at[pl.ds(_EBLK * b, _EBLK), :], sout[p])

    def process(t, p, t2):
        valid = blk(t) < _NFULL
        @pl.when(jnp.logical_and(valid, t2 >= 1))
        def _():
            out_desc(t, p).wait()

        @pl.when(valid)
        def _():
            for d_ in in_descs(t, p):
                d_.wait()

            def ent_body(j, _):
                js = jnp.full((16,), j, jnp.int32)
                for k in range(8):
                    val = plsc.load_gather(v3[p], [avec[k], dvec[k], js])
                    outv[p][j, pl.ds(16 * k, 16)] = val
                return 0

            lax.fori_loop(0, _EBLK, ent_body, 0)
            for d_ in [out_desc(t, p)]:
                d_.start()

        @pl.when(blk(t + 2) < _NFULL)
        def _():
            for d_ in in_descs(t + 2, p):
                d_.start()

    for t0, p0 in ((0, 0), (1, 1)):
        @pl.when(blk(t0) < _NFULL)
        def _(t0=t0, p0=p0):
            for d_ in in_descs(t0, p0):
                d_.start()

    def pair_body(t2, _):
        process(2 * t2 + 0, 0, t2)
        process(2 * t2 + 1, 1, t2)
        return 0

    lax.fori_loop(0, (_TPB + 2) // 2, pair_body, 0)

    for p in range(2):
        pltpu.make_async_copy(
            outv[p], out_h.at[pl.ds(0, _EBLK), :], sout[p]).wait()

    @pl.when(wid == _NW - 1)
    def _():
        pltpu.sync_copy(tail_h, out_a)
        pltpu.sync_copy(out_a, out_h.at[pl.ds(_NFULL * _EBLK, _EBLK), :])


def _score_kernel(subj_h, rel_h, obj_h, entcat_h, relcat_h,
                  out_h,
                  subj_v, rel_v, obj_v,
                  bufs_a, bufs_b, stage_v, out_v, sem_a, sem_b):
    wid = lax.axis_index("s") * _NC + lax.axis_index("c")
    base = wid * _BPW

    for c in range(_NCHUNK):
        pltpu.sync_copy(subj_h.at[pl.ds(base + c * _CHUNK, _CHUNK)], subj_v.at[c])
        pltpu.sync_copy(rel_h.at[pl.ds(base + c * _CHUNK, _CHUNK)], rel_v.at[c])
        pltpu.sync_copy(obj_h.at[pl.ds(base + c * _CHUNK, _CHUNK)], obj_v.at[c])

    bufs = (bufs_a, bufs_b)
    sems = (sem_a, sem_b)

    def fire(c):
        p = c % 2
        s_b, o_b, r_b = bufs[p]
        sem = sems[p]
        return [
            pltpu.async_copy(entcat_h.at[subj_v.at[c]], s_b, sem),
            pltpu.async_copy(entcat_h.at[obj_v.at[c]], o_b, sem),
            pltpu.async_copy(relcat_h.at[rel_v.at[c]], r_b, sem),
        ]

    iota = lax.iota(jnp.int32, 16)
    scatter_idx = iota * _CHUNK
    descs = fire(0)

    for c in range(_NCHUNK):
        next_descs = fire(c + 1) if c + 1 < _NCHUNK else None
        for d_ in descs:
            d_.wait()
        s_b, o_b, r_b = bufs[c % 2]

        def triple_body(i, _, s_b=s_b, o_b=o_b, r_b=r_b):
            acc = jnp.zeros((16,), jnp.float32)
            for k in range(_D // 16):
                re_sl = pl.ds(k * 16, 16)
                im_sl = pl.ds(_D + k * 16, 16)
                a = s_b[i, re_sl]
                b = s_b[i, im_sl]
                x = o_b[i, re_sl]
                y = o_b[i, im_sl]
                p = r_b[i, re_sl]
                q = r_b[i, im_sl]
                u = p * x + q * y
                v = p * y - q * x
                acc = acc + a * u + b * v
            plsc.store_scatter(stage_v, [scatter_idx + i], acc)
            return 0

        lax.fori_loop(0, _CHUNK, triple_body, 0)

        def group_body(g, _, c=c):
            acc = stage_v[pl.ds(g * 16, 16)]
            for k in range(1, 16):
                acc = acc + stage_v[pl.ds(k * _CHUNK + g * 16, 16)]
            out_v[pl.ds(c * _CHUNK + g * 16, 16)] = acc
            return 0

        lax.fori_loop(0, _CHUNK // 16, group_body, 0)
        descs = next_descs

    pltpu.sync_copy(out_v, out_h.at[pl.ds(base, _BPW)])


@jax.jit
def kernel(subj, rel, obj, ent_re, ent_im, rel_re, rel_im):
    re_t = ent_re.T
    im_t = ent_im.T
    relcat = jnp.pad(jnp.concatenate([rel_re, rel_im], axis=1),
                     ((0, 24), (0, 0)))
    tail = jnp.pad(jnp.concatenate([ent_re[_NFULL * _EBLK:],
                                    ent_im[_NFULL * _EBLK:]], axis=1),
                   ((0, _EPAD - _E), (0, 0)))

    fuse = pl.kernel(
        _fuse_kernel,
        out_type=jax.ShapeDtypeStruct((_EPAD, _W), jnp.float32),
        mesh=plsc.VectorSubcoreMesh(**_MESH),
        compiler_params=_PARAMS,
        scratch_types=[
            pltpu.VMEM((16, 8, _EBLK), jnp.float32),
            pltpu.VMEM((16, 8, _EBLK), jnp.float32),
            pltpu.VMEM((_EBLK, _W), jnp.float32),
            pltpu.VMEM((_EBLK, _W), jnp.float32),
            pltpu.SemaphoreType.DMA,
            pltpu.SemaphoreType.DMA,
            pltpu.SemaphoreType.DMA,
            pltpu.SemaphoreType.DMA,
        ],
    )
    entcat = fuse(re_t, im_t, tail)

    row_buf = lambda: pltpu.VMEM((_CHUNK, _W), jnp.float32)
    score = pl.kernel(
        _score_kernel,
        out_type=jax.ShapeDtypeStruct((_B,), jnp.float32),
        mesh=plsc.VectorSubcoreMesh(**_MESH),
        compiler_params=_PARAMS,
        scratch_types=[
            pltpu.VMEM((_NCHUNK, _CHUNK), jnp.int32),
            pltpu.VMEM((_NCHUNK, _CHUNK), jnp.int32),
            pltpu.VMEM((_NCHUNK, _CHUNK), jnp.int32),
            [row_buf() for _ in range(3)],
            [row_buf() for _ in range(3)],
            pltpu.VMEM((16 * _CHUNK,), jnp.float32),
            pltpu.VMEM((_BPW,), jnp.float32),
            pltpu.SemaphoreType.DMA,
            pltpu.SemaphoreType.DMA,
        ],
    )
    return score(subj, rel, obj, entcat, relcat)


# speedup vs baseline: 1.9460x; 1.9460x over previous
import jax
import jax.numpy as jnp
from jax import lax
from jax.experimental import pallas as pl
from jax.experimental.pallas import tpu as pltpu
from jax.experimental.pallas import tpu_sc as plsc

_D = 64
_W = 2 * _D
_B = 16384
_NC = 2
_NS = 16
_NW = _NC * _NS
_BPW = _B // _NW
_CHUNK = 128
_NCHUNK = _BPW // _CHUNK

_E = 100000
_EBLK = 128
_NFULL = _E // _EBLK
_EPAD = (_NFULL + 1) * _EBLK
_TPB = 25

_PARAMS = pltpu.CompilerParams(
    needs_layout_passes=False, use_tc_tiling_on_sc=True)
_MESH = dict(core_axis_name="c", subcore_axis_name="s")


_FBLK = 512


def _fuse_kernel(re_t_ref, im_t_ref, out_ref):
    out_ref[:, 0:_D] = re_t_ref[...].T
    out_ref[:, _D:_W] = im_t_ref[...].T


def _score_kernel(subj_h, rel_h, obj_h, entcat_h, relcat_h,
                  out_h,
                  subj_v, rel_v, obj_v,
                  bufs_a, bufs_b, stage_v, out_v, sem_a, sem_b):
    wid = lax.axis_index("s") * _NC + lax.axis_index("c")
    base = wid * _BPW

    for c in range(_NCHUNK):
        pltpu.sync_copy(subj_h.at[pl.ds(base + c * _CHUNK, _CHUNK)], subj_v.at[c])
        pltpu.sync_copy(rel_h.at[pl.ds(base + c * _CHUNK, _CHUNK)], rel_v.at[c])
        pltpu.sync_copy(obj_h.at[pl.ds(base + c * _CHUNK, _CHUNK)], obj_v.at[c])

    bufs = (bufs_a, bufs_b)
    sems = (sem_a, sem_b)

    def fire(c):
        p = c % 2
        s_b, o_b, r_b = bufs[p]
        sem = sems[p]
        return [
            pltpu.async_copy(entcat_h.at[subj_v.at[c]], s_b, sem),
            pltpu.async_copy(entcat_h.at[obj_v.at[c]], o_b, sem),
            pltpu.async_copy(relcat_h.at[rel_v.at[c]], r_b, sem),
        ]

    iota = lax.iota(jnp.int32, 16)
    scatter_idx = iota * _CHUNK
    descs = fire(0)

    for c in range(_NCHUNK):
        next_descs = fire(c + 1) if c + 1 < _NCHUNK else None
        for d_ in descs:
            d_.wait()
        s_b, o_b, r_b = bufs[c % 2]

        def triple_body(i, _, s_b=s_b, o_b=o_b, r_b=r_b):
            acc = jnp.zeros((16,), jnp.float32)
            for k in range(_D // 16):
                re_sl = pl.ds(k * 16, 16)
                im_sl = pl.ds(_D + k * 16, 16)
                a = s_b[i, re_sl]
                b = s_b[i, im_sl]
                x = o_b[i, re_sl]
                y = o_b[i, im_sl]
                p = r_b[i, re_sl]
                q = r_b[i, im_sl]
                u = p * x + q * y
                v = p * y - q * x
                acc = acc + a * u + b * v
            plsc.store_scatter(stage_v, [scatter_idx + i], acc)
            return 0

        lax.fori_loop(0, _CHUNK, triple_body, 0)

        def group_body(g, _, c=c):
            acc = stage_v[pl.ds(g * 16, 16)]
            for k in range(1, 16):
                acc = acc + stage_v[pl.ds(k * _CHUNK + g * 16, 16)]
            out_v[pl.ds(c * _CHUNK + g * 16, 16)] = acc
            return 0

        lax.fori_loop(0, _CHUNK // 16, group_body, 0)
        descs = next_descs

    pltpu.sync_copy(out_v, out_h.at[pl.ds(base, _BPW)])


@jax.jit
def kernel(subj, rel, obj, ent_re, ent_im, rel_re, rel_im):
    re_t = ent_re.T
    im_t = ent_im.T
    relcat = jnp.pad(jnp.concatenate([rel_re, rel_im], axis=1),
                     ((0, 24), (0, 0)))

    ngrid = (_EPAD + _FBLK - 1) // _FBLK
    entcat = pl.pallas_call(
        _fuse_kernel,
        out_shape=jax.ShapeDtypeStruct((_EPAD, _W), jnp.float32),
        grid=(ngrid,),
        in_specs=[
            pl.BlockSpec((_D, _FBLK), lambda i: (0, i)),
            pl.BlockSpec((_D, _FBLK), lambda i: (0, i)),
        ],
        out_specs=pl.BlockSpec((_FBLK, _W), lambda i: (i, 0)),
    )(re_t, im_t)

    row_buf = lambda: pltpu.VMEM((_CHUNK, _W), jnp.float32)
    score = pl.kernel(
        _score_kernel,
        out_type=jax.ShapeDtypeStruct((_B,), jnp.float32),
        mesh=plsc.VectorSubcoreMesh(**_MESH),
        compiler_params=_PARAMS,
        scratch_types=[
            pltpu.VMEM((_NCHUNK, _CHUNK), jnp.int32),
            pltpu.VMEM((_NCHUNK, _CHUNK), jnp.int32),
            pltpu.VMEM((_NCHUNK, _CHUNK), jnp.int32),
            [row_buf() for _ in range(3)],
            [row_buf() for _ in range(3)],
            pltpu.VMEM((16 * _CHUNK,), jnp.float32),
            pltpu.VMEM((_BPW,), jnp.float32),
            pltpu.SemaphoreType.DMA,
            pltpu.SemaphoreType.DMA,
        ],
    )
    return score(subj, rel, obj, entcat, relcat)


# speedup vs baseline: 2.6009x; 1.3365x over previous
import jax
import jax.numpy as jnp
from jax import lax
from jax.experimental import pallas as pl
from jax.experimental.pallas import tpu as pltpu
from jax.experimental.pallas import tpu_sc as plsc

_D = 64
_W = 2 * _D
_B = 16384
_NC = 2
_NS = 16
_NW = _NC * _NS
_BPW = _B // _NW
_CHUNK = 128
_NCHUNK = _BPW // _CHUNK
_NBUF = 2


def _tec_kernel(subj_h, rel_h, obj_h, entcat_h, relcat_h,
                out_h,
                subj_v, rel_v, obj_v,
                bufs_a, bufs_b, stage_v, out_v, sem_a, sem_b):
    wid = lax.axis_index("s") * _NC + lax.axis_index("c")
    base = wid * _BPW

    for c in range(_NCHUNK):
        pltpu.sync_copy(subj_h.at[pl.ds(base + c * _CHUNK, _CHUNK)], subj_v.at[c])
        pltpu.sync_copy(rel_h.at[pl.ds(base + c * _CHUNK, _CHUNK)], rel_v.at[c])
        pltpu.sync_copy(obj_h.at[pl.ds(base + c * _CHUNK, _CHUNK)], obj_v.at[c])

    bufs = (bufs_a, bufs_b)
    sems = (sem_a, sem_b)

    def fire(c):
        p = c % _NBUF
        s_b, o_b, r_b = bufs[p]
        sem = sems[p]
        return [
            pltpu.async_copy(entcat_h.at[subj_v.at[c]], s_b, sem),
            pltpu.async_copy(entcat_h.at[obj_v.at[c]], o_b, sem),
            pltpu.async_copy(relcat_h.at[rel_v.at[c]], r_b, sem),
        ]

    iota = lax.iota(jnp.int32, 16)
    scatter_idx = iota * _CHUNK
    descs = fire(0)

    for c in range(_NCHUNK):
        next_descs = fire(c + 1) if c + 1 < _NCHUNK else None
        for d_ in descs:
            d_.wait()
        s_b, o_b, r_b = bufs[c % _NBUF]

        def triple_body(i2, _, s_b=s_b, o_b=o_b, r_b=r_b):
            for half in range(2):
                i = i2 * 2 + half
                acc = jnp.zeros((16,), jnp.float32)
                for k in range(_D // 16):
                    re_sl = pl.ds(k * 16, 16)
                    im_sl = pl.ds(_D + k * 16, 16)
                    a = s_b[i, re_sl]
                    b = s_b[i, im_sl]
                    x = o_b[i, re_sl]
                    y = o_b[i, im_sl]
                    p = r_b[i, re_sl]
                    q = r_b[i, im_sl]
                    u = p * x + q * y
                    v = p * y - q * x
                    acc = acc + a * u + b * v
                plsc.store_scatter(stage_v, [scatter_idx + i], acc)
            return 0

        lax.fori_loop(0, _CHUNK // 2, triple_body, 0)

        def group_body(g, _, c=c):
            acc = stage_v[pl.ds(g * 16, 16)]
            for k in range(1, 16):
                acc = acc + stage_v[pl.ds(k * _CHUNK + g * 16, 16)]
            out_v[pl.ds(c * _CHUNK + g * 16, 16)] = acc
            return 0

        lax.fori_loop(0, _CHUNK // 16, group_body, 0)
        descs = next_descs

    pltpu.sync_copy(out_v, out_h.at[pl.ds(base, _BPW)])


@jax.jit
def kernel(subj, rel, obj, ent_re, ent_im, rel_re, rel_im):
    entcat = jnp.concatenate([ent_re, ent_im], axis=1)
    relcat = jnp.concatenate([rel_re, rel_im], axis=1)
    mesh = plsc.VectorSubcoreMesh(core_axis_name="c", subcore_axis_name="s")
    row_buf = lambda: pltpu.VMEM((_CHUNK, _W), jnp.float32)
    run = pl.kernel(
        _tec_kernel,
        out_type=jax.ShapeDtypeStruct((_B,), jnp.float32),
        mesh=mesh,
        compiler_params=pltpu.CompilerParams(
            needs_layout_passes=False, use_tc_tiling_on_sc=True),
        scratch_types=[
            pltpu.VMEM((_NCHUNK, _CHUNK), jnp.int32),
            pltpu.VMEM((_NCHUNK, _CHUNK), jnp.int32),
            pltpu.VMEM((_NCHUNK, _CHUNK), jnp.int32),
            [row_buf() for _ in range(3)],
            [row_buf() for _ in range(3)],
            pltpu.VMEM((16 * _CHUNK,), jnp.float32),
            pltpu.VMEM((_BPW,), jnp.float32),
            pltpu.SemaphoreType.DMA,
            pltpu.SemaphoreType.DMA,
        ],
    )
    return run(subj, rel, obj, entcat, relcat)


# speedup vs baseline: 2.6147x; 1.0053x over previous
import jax
import jax.numpy as jnp
from jax import lax
from jax.experimental import pallas as pl
from jax.experimental.pallas import tpu as pltpu
from jax.experimental.pallas import tpu_sc as plsc

_D = 64
_W = 2 * _D
_B = 16384
_NC = 2
_NS = 16
_NW = _NC * _NS
_BPW = _B // _NW
_CHUNK = 128
_NCHUNK = _BPW // _CHUNK
_NBUF = 2


def _tec_kernel(subj_h, rel_h, obj_h, entcat_h, relcat_h,
                out_h,
                subj_v, rel_v, obj_v,
                bufs_a, bufs_b, stage_v, out_v, sem_a, sem_b):
    wid = lax.axis_index("s") * _NC + lax.axis_index("c")
    base = wid * _BPW

    for c in range(_NCHUNK):
        pltpu.sync_copy(subj_h.at[pl.ds(base + c * _CHUNK, _CHUNK)], subj_v.at[c])
        pltpu.sync_copy(rel_h.at[pl.ds(base + c * _CHUNK, _CHUNK)], rel_v.at[c])
        pltpu.sync_copy(obj_h.at[pl.ds(base + c * _CHUNK, _CHUNK)], obj_v.at[c])

    bufs = (bufs_a, bufs_b)
    sems = (sem_a, sem_b)

    def fire(c):
        p = c % _NBUF
        s_b, o_b, r_b = bufs[p]
        sem = sems[p]
        return [
            pltpu.async_copy(entcat_h.at[subj_v.at[c]], s_b, sem),
            pltpu.async_copy(entcat_h.at[obj_v.at[c]], o_b, sem),
            pltpu.async_copy(relcat_h.at[rel_v.at[c]], r_b, sem),
        ]

    iota = lax.iota(jnp.int32, 16)
    scatter_idx = iota * _CHUNK
    descs = fire(0)

    for c in range(_NCHUNK):
        next_descs = fire(c + 1) if c + 1 < _NCHUNK else None
        for d_ in descs:
            d_.wait()
        s_b, o_b, r_b = bufs[c % _NBUF]

        def triple_body(i, _, s_b=s_b, o_b=o_b, r_b=r_b):
            acc = jnp.zeros((16,), jnp.float32)
            for k in range(_D // 16):
                re_sl = pl.ds(k * 16, 16)
                im_sl = pl.ds(_D + k * 16, 16)
                a = s_b[i, re_sl]
                b = s_b[i, im_sl]
                x = o_b[i, re_sl]
                y = o_b[i, im_sl]
                p = r_b[i, re_sl]
                q = r_b[i, im_sl]
                u = p * x + q * y
                v = p * y - q * x
                acc = acc + a * u + b * v
            plsc.store_scatter(stage_v, [scatter_idx + i], acc)
            return 0

        lax.fori_loop(0, _CHUNK, triple_body, 0)

        def group_body(g, _, c=c):
            acc = stage_v[pl.ds(g * 16, 16)]
            for k in range(1, 16):
                acc = acc + stage_v[pl.ds(k * _CHUNK + g * 16, 16)]
            out_v[pl.ds(c * _CHUNK + g * 16, 16)] = acc
            return 0

        lax.fori_loop(0, _CHUNK // 16, group_body, 0)
        descs = next_descs

    pltpu.sync_copy(out_v, out_h.at[pl.ds(base, _BPW)])


@jax.jit
def kernel(subj, rel, obj, ent_re, ent_im, rel_re, rel_im):
    entcat = jnp.concatenate([ent_re.T, ent_im.T], axis=0).T
    relcat = jnp.concatenate([rel_re, rel_im], axis=1)
    mesh = plsc.VectorSubcoreMesh(core_axis_name="c", subcore_axis_name="s")
    row_buf = lambda: pltpu.VMEM((_CHUNK, _W), jnp.float32)
    run = pl.kernel(
        _tec_kernel,
        out_type=jax.ShapeDtypeStruct((_B,), jnp.float32),
        mesh=mesh,
        compiler_params=pltpu.CompilerParams(
            needs_layout_passes=False, use_tc_tiling_on_sc=True),
        scratch_types=[
            pltpu.VMEM((_NCHUNK, _CHUNK), jnp.int32),
            pltpu.VMEM((_NCHUNK, _CHUNK), jnp.int32),
            pltpu.VMEM((_NCHUNK, _CHUNK), jnp.int32),
            [row_buf() for _ in range(3)],
            [row_buf() for _ in range(3)],
            pltpu.VMEM((16 * _CHUNK,), jnp.float32),
            pltpu.VMEM((_BPW,), jnp.float32),
            pltpu.SemaphoreType.DMA,
            pltpu.SemaphoreType.DMA,
        ],
    )
    return run(subj, rel, obj, entcat, relcat)
